# baseline (speedup 1.0000x reference)
import functools

import jax
import jax.numpy as jnp
from jax import lax
from jax.experimental import pallas as pl
from jax.experimental.pallas import tpu as pltpu
from jax.experimental.pallas import tpu_sc as plsc

N = 10000
E = 640000
D_IN = 128
HID = 16
D_OUT = 128

NW = 32
EB = 100
ROWS = E // EB
RPT = ROWS // NW
RCH = 10
NCH = RPT // RCH


_MESH = plsc.VectorSubcoreMesh(core_axis_name="c", subcore_axis_name="s")
_SC_PARAMS = pltpu.CompilerParams(use_tc_tiling_on_sc=False)


@functools.partial(
    pl.kernel,
    mesh=_MESH,
    out_type=jax.ShapeDtypeStruct((2, N, HID), jnp.float32),
    scratch_types=[
        pltpu.VMEM((RPT, EB), jnp.int32),
        pltpu.VMEM((RPT, EB), jnp.int32),
        pltpu.VMEM((2, RCH, EB, HID), jnp.float32),
        pltpu.VMEM_SHARED((N, HID), jnp.float32),
        pltpu.VMEM_SHARED((N, HID), jnp.float32),
        pltpu.SemaphoreType.DMA,
        pltpu.SemaphoreType.DMA,
        pltpu.SemaphoreType.DMA,
        pltpu.SemaphoreType.DMA,
        pltpu.SemaphoreType.DMA,
    ],
    compiler_params=_SC_PARAMS,
)
def _agg_kernel(hs_hbm, src_hbm, dst_hbm, zeros_hbm, out_hbm,
                idx_s, idx_d, rows, hs_s, acc,
                isem, gsem0, gsem1, ssem0, ssem1):
    c = lax.axis_index("c")
    s = lax.axis_index("s")
    wid = s * 2 + c
    ic0 = pltpu.async_copy(src_hbm.at[pl.ds(wid * RPT, RPT)], idx_s, isem)
    ic1 = pltpu.async_copy(dst_hbm.at[pl.ds(wid * RPT, RPT)], idx_d, isem)

    @pl.when(s == 0)
    def _():
        pltpu.sync_copy(hs_hbm, hs_s)

    @pl.when(s == 1)
    def _():
        pltpu.sync_copy(zeros_hbm, acc)

    plsc.subcore_barrier()
    ic0.wait()
    ic1.wait()
    gsems = (gsem0, gsem1)
    ssems = (ssem0, ssem1)

    def body(g2, carry):
        for b in range(2):
            ch = g2 * 2 + b

            @pl.when(ch >= 2)
            def _():
                for j in range(RCH):
                    pltpu.make_async_copy(
                        rows.at[b, j], acc.at[idx_d.at[0]], ssems[b]).wait()

            gcs = [pltpu.async_copy(
                       hs_s.at[idx_s.at[ch * RCH + j]], rows.at[b, j],
                       gsems[b])
                   for j in range(RCH)]
            for j in range(RCH):
                gcs[j].wait()
                pltpu.async_copy(
                    rows.at[b, j], acc.at[idx_d.at[ch * RCH + j]], ssems[b],
                    add=True)
        return carry

    lax.fori_loop(0, NCH // 2, body, 0)
    for b in range(2):
        for j in range(RCH):
            pltpu.make_async_copy(
                rows.at[b, j], acc.at[idx_d.at[0]], ssems[b]).wait()
    plsc.subcore_barrier()

    @pl.when(s == 0)
    def _():
        pltpu.sync_copy(acc, out_hbm.at[c])


RW = N // 8

BN = 1000
GRID = N // BN


def _h1_body(x_ref, w1_ref, h1_ref):
    h1_ref[...] = jnp.dot(x_ref[...], w1_ref[...],
                          preferred_element_type=jnp.float32)


_h1_call = pl.pallas_call(
    _h1_body,
    grid=(GRID,),
    in_specs=[
        pl.BlockSpec((BN, D_IN), lambda i: (i, 0)),
        pl.BlockSpec((D_IN, HID), lambda i: (0, 0)),
    ],
    out_specs=pl.BlockSpec((BN, HID), lambda i: (i, 0)),
    out_shape=jax.ShapeDtypeStruct((N, HID), jnp.float32),
)


def _prepw_body(h1_ref, d0_ref, d1_ref, hs1_ref, dis_ref):
    dis = lax.rsqrt(d0_ref[...] + d1_ref[...] + 1.0)
    hs1_ref[...] = h1_ref[...] * dis
    dis_ref[...] = dis


_prepw_call = pl.pallas_call(
    _prepw_body,
    out_shape=[
        jax.ShapeDtypeStruct((RW, 128), jnp.float32),
        jax.ShapeDtypeStruct((RW, 128), jnp.float32),
    ],
)


def _midw_body(p0_ref, p1_ref, hs1_ref, dis_ref, b1_ref, hs2_ref):
    dis = dis_ref[...]
    t = (p0_ref[...] + p1_ref[...] + hs1_ref[...]) * dis + b1_ref[...]
    hs2_ref[...] = jnp.maximum(t, 0.0) * dis


_midw_call = pl.pallas_call(
    _midw_body,
    out_shape=jax.ShapeDtypeStruct((RW, 128), jnp.float32),
)


def _tw_body(q0_ref, q1_ref, hs2_ref, dis_ref, t_ref):
    t_ref[...] = (q0_ref[...] + q1_ref[...] + hs2_ref[...]) * dis_ref[...]


_tw_call = pl.pallas_call(
    _tw_body,
    out_shape=jax.ShapeDtypeStruct((RW, 128), jnp.float32),
)


def _out_body(t_ref, w2_ref, b2_ref, o_ref):
    h = jnp.dot(t_ref[...], w2_ref[...],
                preferred_element_type=jnp.float32) + b2_ref[...]
    m = jnp.max(h, axis=1, keepdims=True)
    ex = jnp.exp(h - m)
    ssum = jnp.sum(ex, axis=1, keepdims=True)
    o_ref[...] = (h - m) - jnp.log(ssum)


_out_call = pl.pallas_call(
    _out_body,
    grid=(GRID,),
    in_specs=[
        pl.BlockSpec((BN, HID), lambda i: (i, 0)),
        pl.BlockSpec((HID, D_OUT), lambda i: (0, 0)),
        pl.BlockSpec((1, D_OUT), lambda i: (0, 0)),
    ],
    out_specs=pl.BlockSpec((BN, D_OUT), lambda i: (i, 0)),
    out_shape=jax.ShapeDtypeStruct((N, D_OUT), jnp.float32),
)


def _wide(a):
    return jnp.reshape(a, (RW, 128))


def kernel(traffic, path_to_queue, W1, b1, W2, b2):
    src2d = path_to_queue[0].reshape(ROWS, EB)
    dst2d = path_to_queue[1].reshape(ROWS, EB)
    zeros2d = jnp.zeros((N, HID), jnp.float32)
    ones2d = jnp.ones((N, HID), jnp.float32)
    b1w = jnp.tile(b1.reshape(1, HID), (1, 8))

    d = _agg_kernel(ones2d, dst2d, dst2d, zeros2d)
    h1 = _h1_call(traffic, W1)
    hs1w, disw = _prepw_call(_wide(h1), _wide(d[0]), _wide(d[1]))

    p = _agg_kernel(hs1w.reshape(N, HID), src2d, dst2d, zeros2d)
    hs2w = _midw_call(_wide(p[0]), _wide(p[1]), hs1w, disw, b1w)

    q = _agg_kernel(hs2w.reshape(N, HID), src2d, dst2d, zeros2d)
    tw = _tw_call(_wide(q[0]), _wide(q[1]), hs2w, disw)
    return _out_call(tw.reshape(N, HID), W2, b2.reshape(1, D_OUT))


# speedup vs baseline: 1.0298x; 1.0298x over previous
import functools

import jax
import jax.numpy as jnp
from jax import lax
from jax.experimental import pallas as pl
from jax.experimental.pallas import tpu as pltpu
from jax.experimental.pallas import tpu_sc as plsc

N = 10000
E = 640000
D_IN = 128
HID = 16
D_OUT = 128

NW = 32
EB = 100
ROWS = E // EB
RPT = ROWS // NW
RCH = 10
NCH = RPT // RCH
ARCH = 20
ANCH = RPT // ARCH


_MESH = plsc.VectorSubcoreMesh(core_axis_name="c", subcore_axis_name="s")
_SC_PARAMS = pltpu.CompilerParams(use_tc_tiling_on_sc=False)


@functools.partial(
    pl.kernel,
    mesh=_MESH,
    out_type=jax.ShapeDtypeStruct((2, N, HID), jnp.float32),
    scratch_types=[
        pltpu.VMEM((RPT, EB), jnp.int32),
        pltpu.VMEM((EB, HID), jnp.float32),
        pltpu.VMEM_SHARED((N, HID), jnp.float32),
        pltpu.SemaphoreType.DMA,
        pltpu.SemaphoreType.DMA,
        pltpu.SemaphoreType.DMA,
    ],
    compiler_params=_SC_PARAMS,
)
def _deg_kernel(dst_hbm, zeros_hbm, ones_hbm, out_hbm, idxd, onesv, accd,
                isem, ssem0, ssem1):
    c = lax.axis_index("c")
    s = lax.axis_index("s")
    wid = s * 2 + c
    icp = pltpu.async_copy(dst_hbm.at[pl.ds(wid * RPT, RPT)], idxd, isem)
    pltpu.sync_copy(ones_hbm, onesv)

    @pl.when(s == 0)
    def _():
        pltpu.sync_copy(zeros_hbm, accd)

    plsc.subcore_barrier()
    icp.wait()
    ssems = (ssem0, ssem1)

    def body(g2, carry):
        for b in range(2):
            ch = g2 * 2 + b

            @pl.when(ch >= 2)
            def _():
                for j in range(RCH):
                    pltpu.make_async_copy(
                        onesv, accd.at[idxd.at[0]], ssems[b]).wait()

            for j in range(RCH):
                pltpu.async_copy(
                    onesv, accd.at[idxd.at[ch * RCH + j]], ssems[b],
                    add=True)
        return carry

    lax.fori_loop(0, NCH // 2, body, 0)
    for b in range(2):
        for j in range(RCH):
            pltpu.make_async_copy(onesv, accd.at[idxd.at[0]], ssems[b]).wait()
    plsc.subcore_barrier()

    @pl.when(s == 0)
    def _():
        pltpu.sync_copy(accd, out_hbm.at[c])


@functools.partial(
    pl.kernel,
    mesh=_MESH,
    out_type=jax.ShapeDtypeStruct((2, N, HID), jnp.float32),
    scratch_types=[
        pltpu.VMEM((RPT, EB), jnp.int32),
        pltpu.VMEM((RPT, EB), jnp.int32),
        pltpu.VMEM((2, ARCH, EB, HID), jnp.float32),
        pltpu.VMEM_SHARED((N, HID), jnp.float32),
        pltpu.VMEM_SHARED((N, HID), jnp.float32),
        pltpu.SemaphoreType.DMA,
        pltpu.SemaphoreType.DMA,
        pltpu.SemaphoreType.DMA,
        pltpu.SemaphoreType.DMA,
        pltpu.SemaphoreType.DMA,
    ],
    compiler_params=_SC_PARAMS,
)
def _agg_kernel(hs_hbm, src_hbm, dst_hbm, zeros_hbm, out_hbm,
                idx_s, idx_d, rows, hs_s, acc,
                isem, gsem0, gsem1, ssem0, ssem1):
    c = lax.axis_index("c")
    s = lax.axis_index("s")
    wid = s * 2 + c
    ic0 = pltpu.async_copy(src_hbm.at[pl.ds(wid * RPT, RPT)], idx_s, isem)
    ic1 = pltpu.async_copy(dst_hbm.at[pl.ds(wid * RPT, RPT)], idx_d, isem)

    @pl.when(s == 0)
    def _():
        pltpu.sync_copy(hs_hbm, hs_s)

    @pl.when(s == 1)
    def _():
        pltpu.sync_copy(zeros_hbm, acc)

    plsc.subcore_barrier()
    ic0.wait()
    ic1.wait()
    gsems = (gsem0, gsem1)
    ssems = (ssem0, ssem1)

    def body(g2, carry):
        for b in range(2):
            ch = g2 * 2 + b

            @pl.when(ch >= 2)
            def _():
                for j in range(ARCH):
                    pltpu.make_async_copy(
                        rows.at[b, j], acc.at[idx_d.at[0]], ssems[b]).wait()

            gcs = [pltpu.async_copy(
                       hs_s.at[idx_s.at[ch * ARCH + j]], rows.at[b, j],
                       gsems[b])
                   for j in range(ARCH)]
            for j in range(ARCH):
                gcs[j].wait()
                pltpu.async_copy(
                    rows.at[b, j], acc.at[idx_d.at[ch * ARCH + j]], ssems[b],
                    add=True)
        return carry

    lax.fori_loop(0, ANCH // 2, body, 0)
    for b in range(2):
        for j in range(ARCH):
            pltpu.make_async_copy(
                rows.at[b, j], acc.at[idx_d.at[0]], ssems[b]).wait()
    plsc.subcore_barrier()

    @pl.when(s == 0)
    def _():
        pltpu.sync_copy(acc, out_hbm.at[c])


BN = 1000
GRID = N // BN


def _prep_body(x_ref, w1_ref, d0_ref, d1_ref, hs1_ref, dis_ref):
    deg = d0_ref[:, :1] + d1_ref[:, :1] + 1.0
    dis = lax.rsqrt(deg)
    h1 = jnp.dot(x_ref[...], w1_ref[...], preferred_element_type=jnp.float32)
    hs1_ref[...] = h1 * dis
    dis_ref[...] = dis


_prep_call = pl.pallas_call(
    _prep_body,
    grid=(GRID,),
    in_specs=[
        pl.BlockSpec((BN, D_IN), lambda i: (i, 0)),
        pl.BlockSpec((D_IN, HID), lambda i: (0, 0)),
        pl.BlockSpec((BN, HID), lambda i: (i, 0)),
        pl.BlockSpec((BN, HID), lambda i: (i, 0)),
    ],
    out_specs=[
        pl.BlockSpec((BN, HID), lambda i: (i, 0)),
        pl.BlockSpec((BN, 1), lambda i: (i, 0)),
    ],
    out_shape=[
        jax.ShapeDtypeStruct((N, HID), jnp.float32),
        jax.ShapeDtypeStruct((N, 1), jnp.float32),
    ],
)


def _mid_body(p0_ref, p1_ref, hs1_ref, dis_ref, b1_ref, hs2_ref):
    dis = dis_ref[...]
    t = (p0_ref[...] + p1_ref[...] + hs1_ref[...]) * dis + b1_ref[...]
    hs2_ref[...] = jnp.maximum(t, 0.0) * dis


_mid_call = pl.pallas_call(
    _mid_body,
    grid=(GRID,),
    in_specs=[
        pl.BlockSpec((BN, HID), lambda i: (i, 0)),
        pl.BlockSpec((BN, HID), lambda i: (i, 0)),
        pl.BlockSpec((BN, HID), lambda i: (i, 0)),
        pl.BlockSpec((BN, 1), lambda i: (i, 0)),
        pl.BlockSpec((1, HID), lambda i: (0, 0)),
    ],
    out_specs=pl.BlockSpec((BN, HID), lambda i: (i, 0)),
    out_shape=jax.ShapeDtypeStruct((N, HID), jnp.float32),
)


def _out_body(q0_ref, q1_ref, hs2_ref, dis_ref, w2_ref, b2_ref, o_ref):
    t = (q0_ref[...] + q1_ref[...] + hs2_ref[...]) * dis_ref[...]
    h = jnp.dot(t, w2_ref[...], preferred_element_type=jnp.float32) + b2_ref[...]
    m = jnp.max(h, axis=1, keepdims=True)
    ex = jnp.exp(h - m)
    ssum = jnp.sum(ex, axis=1, keepdims=True)
    o_ref[...] = (h - m) - jnp.log(ssum)


_out_call = pl.pallas_call(
    _out_body,
    grid=(GRID,),
    in_specs=[
        pl.BlockSpec((BN, HID), lambda i: (i, 0)),
        pl.BlockSpec((BN, HID), lambda i: (i, 0)),
        pl.BlockSpec((BN, HID), lambda i: (i, 0)),
        pl.BlockSpec((BN, 1), lambda i: (i, 0)),
        pl.BlockSpec((HID, D_OUT), lambda i: (0, 0)),
        pl.BlockSpec((1, D_OUT), lambda i: (0, 0)),
    ],
    out_specs=pl.BlockSpec((BN, D_OUT), lambda i: (i, 0)),
    out_shape=jax.ShapeDtypeStruct((N, D_OUT), jnp.float32),
)


def kernel(traffic, path_to_queue, W1, b1, W2, b2):
    src2d = path_to_queue[0].reshape(ROWS, EB)
    dst2d = path_to_queue[1].reshape(ROWS, EB)
    zeros2d = jnp.zeros((N, HID), jnp.float32)
    onese = jnp.ones((EB, HID), jnp.float32)

    d = _deg_kernel(dst2d, zeros2d, onese)
    hs1, dis = _prep_call(traffic, W1, d[0], d[1])

    p = _agg_kernel(hs1, src2d, dst2d, zeros2d)
    hs2 = _mid_call(p[0], p[1], hs1, dis, b1.reshape(1, HID))

    q = _agg_kernel(hs2, src2d, dst2d, zeros2d)
    return _out_call(q[0], q[1], hs2, dis, W2, b2.reshape(1, D_OUT))
